# baseline (speedup 1.0000x reference)
import numpy as np

import jax
import jax.numpy as jnp
from jax.experimental import pallas as pl
from jax.experimental.pallas import tpu as pltpu
from jax.sharding import Mesh, PartitionSpec as P

_BLOCK_N = 256
_BF16 = jnp.bfloat16


def _shift_rows(a, di):
    if di == 0:
        return a
    pad = jnp.zeros((di, a.shape[1]), a.dtype)
    return jnp.concatenate([a[di:], pad], axis=0)


def _fused_kernel(x_ref, m1c_ref, b1c_ref, m2c_ref, b2c_ref, w1f_ref,
                  b1f_ref, w2p_ref, b2f_ref, w3p_ref, b3f_ref, o_ref):
    f32 = jnp.float32
    x = x_ref[...].astype(_BF16)
    M = x.shape[0]

    xc = jnp.concatenate([_shift_rows(x, di) for di in range(5)], axis=1)
    z1 = jnp.dot(xc, m1c_ref[...], preferred_element_type=f32)
    r1 = jnp.maximum(z1 + b1c_ref[...], 0.0).astype(_BF16)

    h1 = jnp.maximum(r1[:, :128], r1[:, 128:])
    q1 = h1.reshape(M // 2, 256)
    p1 = jnp.maximum(q1[:, :128], q1[:, 128:])

    pc = jnp.concatenate([_shift_rows(p1, di) for di in range(5)], axis=1)
    z2 = jnp.dot(pc, m2c_ref[...], preferred_element_type=f32)
    r2 = jnp.maximum(z2 + b2c_ref[...], 0.0).astype(_BF16)

    h2 = jnp.maximum(r2[:, :128], r2[:, 128:])
    q2 = h2.reshape(M // 4, 256)
    p2 = jnp.maximum(q2[:, :128], q2[:, 128:])

    fv = p2.reshape(M // 32, 1024)
    f = jnp.maximum(jnp.dot(fv, w1f_ref[...], preferred_element_type=f32)
                    + b1f_ref[...], 0.0).astype(_BF16)

    f = jnp.maximum(jnp.dot(f, w2p_ref[...], preferred_element_type=f32)
                    + b2f_ref[...], 0.0).astype(_BF16)
    o = jnp.dot(f, w3p_ref[...], preferred_element_type=f32) + b3f_ref[...]
    o_ref[...] = o[:, :10]


def _forward(xf, m1c, b1c, m2c, b2c, w1f, b1f, w2b, b2f, w3b, b3f):
    B = xf.shape[0] // 32
    N = _BLOCK_N if B % _BLOCK_N == 0 else B
    M = N * 32

    full2d = lambda a: pl.BlockSpec(a.shape, lambda b: (0, 0))
    macs_blk = (M * 160 * 256 + (M // 2) * 640 * 256 + N * 1024 * 128
                + 2 * N * 128 * 128)
    return pl.pallas_call(
        _fused_kernel,
        out_shape=jax.ShapeDtypeStruct((B, 10), jnp.float32),
        grid=(B // N,),
        in_specs=[pl.BlockSpec((M, 32), lambda b: (b, 0)),
                  full2d(m1c), full2d(b1c), full2d(m2c), full2d(b2c),
                  full2d(w1f), full2d(b1f), full2d(w2b),
                  full2d(b2f), full2d(w3b), full2d(b3f)],
        out_specs=pl.BlockSpec((N, 10), lambda b: (b, 0)),
        compiler_params=pltpu.CompilerParams(
            dimension_semantics=("parallel",),
            vmem_limit_bytes=64 * 1024 * 1024),
        cost_estimate=pl.CostEstimate(
            flops=2 * macs_blk * (B // N), transcendentals=0,
            bytes_accessed=2 * B * 32 * 32 + 4 * B * 128),
    )(xf, m1c, b1c, m2c, b2c, w1f, b1f, w2b, b2f, w3b, b3f)


def kernel(m1, b1c, m2, b2c, w1r, b1f, w2p, b2f, w3p, b3f, x_nchw):
    B = x_nchw.shape[0]
    xf = x_nchw.reshape(B * 32, 32)
    m1c = m1.reshape(160, 256).astype(_BF16)
    m2c = m2.reshape(640, 256).astype(_BF16)
    w1f = jnp.concatenate(
        [w1r, jnp.zeros((3, 128, 128), jnp.float32)], axis=0
    ).reshape(1024, 128).astype(_BF16)
    w2b = w2p.astype(_BF16)
    w3b = w3p.astype(_BF16)
    args = (xf, m1c, b1c, m2c, b2c, w1f, b1f, w2b, b2f, w3b, b3f)

    devs = jax.devices()
    n_dev = 2 if (len(devs) >= 2 and B % (2 * _BLOCK_N) == 0) else 1
    if n_dev == 1:
        return _forward(*args)
    mesh = Mesh(np.array(devs[:2]), ("d",))
    specs = (P("d"),) + (P(),) * 10
    fwd = jax.shard_map(_forward, mesh=mesh, in_specs=specs,
                        out_specs=P("d"), check_vma=False)
    return fwd(*args)


# speedup vs baseline: 3.0578x; 3.0578x over previous
import jax
import jax.numpy as jnp
from jax.experimental import pallas as pl
from jax.experimental.pallas import tpu as pltpu

_BLOCK_N = 512
_BF16 = jnp.bfloat16


def _shift_rows(a, di):
    if di == 0:
        return a
    pad = jnp.zeros((di, a.shape[1]), a.dtype)
    return jnp.concatenate([a[di:], pad], axis=0)


def _fused_kernel(x_ref, m1c_ref, b1c_ref, m2c_ref, b2c_ref, w1f_ref,
                  b1f_ref, w2p_ref, b2f_ref, w3p_ref, b3f_ref, o_ref):
    f32 = jnp.float32
    x = x_ref[...].astype(_BF16)
    M = x.shape[0]

    xc = jnp.concatenate([_shift_rows(x, di) for di in range(5)], axis=1)
    z1 = jnp.dot(xc, m1c_ref[...], preferred_element_type=f32)
    r1 = jnp.maximum(z1 + b1c_ref[...], 0.0).astype(_BF16)

    h1 = jnp.maximum(r1[:, :128], r1[:, 128:])
    q1 = h1.reshape(M // 2, 256)
    p1 = jnp.maximum(q1[:, :128], q1[:, 128:])

    pc = jnp.concatenate([_shift_rows(p1, di) for di in range(5)], axis=1)
    z2 = jnp.dot(pc, m2c_ref[...], preferred_element_type=f32)
    r2 = jnp.maximum(z2 + b2c_ref[...], 0.0).astype(_BF16)

    h2 = jnp.maximum(r2[:, :128], r2[:, 128:])
    q2 = h2.reshape(M // 4, 256)
    p2 = jnp.maximum(q2[:, :128], q2[:, 128:])

    fv = p2.reshape(M // 32, 1024)
    f = jnp.maximum(jnp.dot(fv, w1f_ref[...], preferred_element_type=f32)
                    + b1f_ref[...], 0.0).astype(_BF16)

    f = jnp.maximum(jnp.dot(f, w2p_ref[...], preferred_element_type=f32)
                    + b2f_ref[...], 0.0).astype(_BF16)
    o = jnp.dot(f, w3p_ref[...], preferred_element_type=f32) + b3f_ref[...]
    o_ref[...] = o[:, :10]


def _forward(xf, m1c, b1c, m2c, b2c, w1f, b1f, w2b, b2f, w3b, b3f):
    B = xf.shape[0] // 32
    N = _BLOCK_N if B % _BLOCK_N == 0 else B
    M = N * 32

    full2d = lambda a: pl.BlockSpec(a.shape, lambda b: (0, 0))
    macs_blk = (M * 160 * 256 + (M // 2) * 640 * 256 + N * 1024 * 128
                + 2 * N * 128 * 128)
    return pl.pallas_call(
        _fused_kernel,
        out_shape=jax.ShapeDtypeStruct((B, 10), jnp.float32),
        grid=(B // N,),
        in_specs=[pl.BlockSpec((M, 32), lambda b: (b, 0)),
                  full2d(m1c), full2d(b1c), full2d(m2c), full2d(b2c),
                  full2d(w1f), full2d(b1f), full2d(w2b),
                  full2d(b2f), full2d(w3b), full2d(b3f)],
        out_specs=pl.BlockSpec((N, 10), lambda b: (b, 0)),
        compiler_params=pltpu.CompilerParams(
            dimension_semantics=("parallel",),
            vmem_limit_bytes=64 * 1024 * 1024),
        cost_estimate=pl.CostEstimate(
            flops=2 * macs_blk * (B // N), transcendentals=0,
            bytes_accessed=2 * B * 32 * 32 + 4 * B * 128),
    )(xf, m1c, b1c, m2c, b2c, w1f, b1f, w2b, b2f, w3b, b3f)


def kernel(m1, b1c, m2, b2c, w1r, b1f, w2p, b2f, w3p, b3f, x_nchw):
    B = x_nchw.shape[0]
    xf = x_nchw.reshape(B * 32, 32)
    m1c = m1.reshape(160, 256).astype(_BF16)
    m2c = m2.reshape(640, 256).astype(_BF16)
    w1f = jnp.concatenate(
        [w1r, jnp.zeros((3, 128, 128), jnp.float32)], axis=0
    ).reshape(1024, 128).astype(_BF16)
    w2b = w2p.astype(_BF16)
    w3b = w3p.astype(_BF16)
    return _forward(xf, m1c, b1c, m2c, b2c, w1f, b1f, w2b, b2f, w3b, b3f)
